# baseline (speedup 1.0000x reference)
import functools

import jax
import jax.numpy as jnp
from jax import lax
from jax.experimental import pallas as pl
from jax.experimental.pallas import tpu as pltpu
from jax.experimental.pallas import tpu_sc as plsc

_B = 16384
_NCONT = 13
_D = 32
_F = 26
_NR = _NCONT + _F
_V = 100001
_VA = 99968
_VT = _V - _VA
_VROW = 100016
_TBL = _F * _D * _VROW

_CB = 2048
_NCHUNK = _B // _CB
_NSLAB = _F * (_D // 8)


_CHW = 1408
_NCH = _VA // _CHW
_GRP = 4
_NGRP = 18
_AROW = _GRP * _CHW + 40
_ABUF = 8 * _AROW


def _make_reformat_kernel():
    mesh = plsc.VectorSubcoreMesh(core_axis_name="c", subcore_axis_name="s")

    @functools.partial(
        pl.kernel,
        mesh=mesh,
        out_type=jax.ShapeDtypeStruct((_TBL,), jnp.float32),
        compiler_params=pltpu.CompilerParams(use_tc_tiling_on_sc=True),
        scratch_types=[
            pltpu.VMEM((2, 8, _CHW), jnp.float32),
            pltpu.VMEM((2 * _ABUF,), jnp.float32),
            pltpu.VMEM((8, 48), jnp.float32),
            pltpu.SemaphoreType.DMA,
            pltpu.SemaphoreType.DMA,
            pltpu.SemaphoreType.DMA,
        ],
    )
    def reformat(tab_hbm, tail_hbm, flat_hbm, chk_v, asm_v, tl_v,
                 sem_r, sem_w, sem_t):
        wid = lax.axis_index("s") * 2 + lax.axis_index("c")

        def dsw_chunk(par, cbase):
            def dsw(t, c2):
                k0 = t * 4
                for kk in range(4):
                    rbase = cbase + (k0 + kk) * _AROW
                    for j in range(11):
                        for i in range(8):
                            col = j * 128 + i * 16
                            asm_v[pl.ds(rbase + col, 16)] = (
                                chk_v[par, k0 + kk, pl.ds(col, 16)])
                return c2

            lax.fori_loop(0, 2, dsw, 0)

        def drain_writes(n, width):
            for _ in range(n):
                pltpu.make_async_copy(
                    asm_v.at[pl.ds(0, width)],
                    flat_hbm.at[pl.ds(0, width)], sem_w).wait()

        def half_run(s, start_chunk, nfull, with_tail):
            f = s // 4
            d0 = pl.multiple_of((s % 4) * 8, 8)
            base = (f * _D + d0) * _VROW
            end_chunk = start_chunk + nfull * 4 + (3 if with_tail else 0)

            def fetch(c):
                return pltpu.async_copy(
                    tab_hbm.at[f, pl.ds(d0, 8), pl.ds(c * _CHW, _CHW)],
                    chk_v.at[c % 2], sem_r)

            if with_tail:
                tail_cp = pltpu.async_copy(
                    tail_hbm.at[f, pl.ds(d0, 8), pl.ds(0, 48)], tl_v,
                    sem_t)
            fetch(start_chunk)

            def cloop(c2, carry):
                c = start_chunk + c2

                @pl.when(c + 1 < end_chunk)
                def _pf():
                    pltpu.async_copy(
                        tab_hbm.at[f, pl.ds(d0, 8),
                                   pl.ds((c + 1) * _CHW, _CHW)],
                        chk_v.at[(c + 1) % 2], sem_r)

                pltpu.make_async_copy(
                    tab_hbm.at[f, pl.ds(d0, 8), pl.ds(0, _CHW)],
                    chk_v.at[c % 2], sem_r).wait()
                cc = c2 % 4
                g_rel = c2 // 4
                abase = (g_rel % 2) * _ABUF

                @pl.when((cc == 0) & (c2 >= 8))
                def _dw():
                    drain_writes(8, _GRP * _CHW)

                dsw_chunk(c % 2, abase + cc * 11 * 128)

                @pl.when(cc == 3)
                def _wr():
                    gbase = base + (start_chunk + g_rel * 4) * _CHW
                    for k in range(8):
                        pltpu.async_copy(
                            asm_v.at[pl.ds(abase + k * _AROW,
                                           _GRP * _CHW)],
                            flat_hbm.at[pl.ds(gbase + k * _VROW,
                                              _GRP * _CHW)], sem_w)

                return carry

            lax.fori_loop(0, nfull * 4, cloop, 0)
            drain_writes(16, _GRP * _CHW)
            if with_tail:
                tstart = start_chunk + nfull * 4

                def tloop(t, carry):
                    c = tstart + t

                    @pl.when(c + 1 < end_chunk)
                    def _pf():
                        pltpu.async_copy(
                            tab_hbm.at[f, pl.ds(d0, 8),
                                       pl.ds((c + 1) * _CHW, _CHW)],
                            chk_v.at[(c + 1) % 2], sem_r)

                    pltpu.make_async_copy(
                        tab_hbm.at[f, pl.ds(d0, 8), pl.ds(0, _CHW)],
                        chk_v.at[c % 2], sem_r).wait()
                    dsw_chunk(c % 2, t * 11 * 128)
                    return carry

                lax.fori_loop(0, 3, tloop, 0)
                tail_cp.wait()
                for k in range(8):
                    for i in range(3):
                        asm_v[pl.ds(k * _AROW + 3 * _CHW + i * 16, 16)] = (
                            tl_v[k, pl.ds(i * 16, 16)])
                twidth = 3 * _CHW + 40
                gbase = base + tstart * _CHW
                for k in range(8):
                    pltpu.async_copy(
                        asm_v.at[pl.ds(k * _AROW, twidth)],
                        flat_hbm.at[pl.ds(gbase + k * _VROW, twidth)],
                        sem_w)
                drain_writes(8, twidth)

        def slab_loop(m, carry):
            item = wid + 32 * m

            @pl.when(item < 2 * _NSLAB)
            def _go():
                s = item // 2

                @pl.when(item % 2 == 0)
                def _lo():
                    half_run(s, 0, 9, False)

                @pl.when(item % 2 == 1)
                def _hi():
                    half_run(s, 36, 8, True)

            return carry

        lax.fori_loop(0, 7, slab_loop, 0)

    return reformat


def _make_lookup_kernel():
    mesh = plsc.VectorSubcoreMesh(core_axis_name="c", subcore_axis_name="s")

    @functools.partial(
        pl.kernel,
        mesh=mesh,
        out_type=jax.ShapeDtypeStruct((_NR, _D, _B), jnp.float32),
        compiler_params=pltpu.CompilerParams(
            use_tc_tiling_on_sc=False, needs_layout_passes=False),
        scratch_types=[
            pltpu.VMEM((_VROW + 8,), jnp.float32),
            pltpu.VMEM((2, _CB), jnp.int32),
            pltpu.VMEM((2, _CB), jnp.float32),
            pltpu.VMEM((2, _CB), jnp.float32),
            pltpu.VMEM((16,), jnp.float32),
            pltpu.VMEM((16,), jnp.float32),
            pltpu.SemaphoreType.DMA,
            pltpu.SemaphoreType.DMA,
            pltpu.SemaphoreType.DMA,
        ],
    )
    def lookup(cat_hbm, x_hbm, w_hbm, c_hbm, flat_hbm, out_hbm,
               vec_v, cb_v, xb_v, ob_v, w_v, c_v,
               sem_v, sem_i, sem_o):
        wid = lax.axis_index("s") * 2 + lax.axis_index("c")

        pltpu.sync_copy(w_hbm.at[wid], w_v)
        pltpu.sync_copy(c_hbm.at[wid], c_v)
        wv = w_v[pl.ds(0, 16)]
        cv = c_v[pl.ds(0, 16)]

        xrow = pltpu.async_copy(x_hbm.at[0], vec_v.at[pl.ds(0, _B)], sem_i)

        def cont_row(n):
            wn = wv[n]
            cn = cv[n]
            xoff = (n % 2) * _B
            copies = []
            for k in range(_NCHUNK):
                par = k % 2

                def fma(j, c2):
                    for u in range(4):
                        off = j * 64 + u * 16
                        ob_v[par, pl.ds(off, 16)] = (
                            vec_v[pl.ds(xoff + k * _CB + off, 16)] * wn + cn)
                    return c2

                lax.fori_loop(0, _CB // 64, fma, 0)
                if len(copies) == 2:
                    copies.pop(0).wait()
                copies.append(pltpu.async_copy(
                    ob_v.at[par],
                    out_hbm.at[n, wid, pl.ds(k * _CB, _CB)], sem_o))
            for cp in copies:
                cp.wait()

        for n in range(_NCONT):
            xrow.wait()
            if n + 1 < _NCONT:
                xrow = pltpu.async_copy(
                    x_hbm.at[n + 1],
                    vec_v.at[pl.ds(((n + 1) % 2) * _B, _B)], sem_i)
            cont_row(n)

        def cat_row(f, _):
            vec_cp = pltpu.async_copy(
                flat_hbm.at[pl.ds((f * _D + wid) * _VROW, _VROW)],
                vec_v.at[pl.ds(0, _VROW)], sem_v)
            pltpu.sync_copy(cat_hbm.at[f, pl.ds(0, _CB)], cb_v.at[0])
            vec_cp.wait()
            copies = []
            for k in range(_NCHUNK):
                par = k % 2
                if k + 1 < _NCHUNK:
                    nxt = pltpu.async_copy(
                        cat_hbm.at[f, pl.ds((k + 1) * _CB, _CB)],
                        cb_v.at[1 - par], sem_i)

                def gath(j, c2):
                    for u in range(8):
                        sl = pl.ds(j * 128 + u * 16, 16)
                        idx = cb_v[par, sl]
                        ob_v[par, sl] = plsc.load_gather(vec_v, [idx])
                    return c2

                lax.fori_loop(0, _CB // 128, gath, 0)
                if len(copies) == 2:
                    copies.pop(0).wait()
                copies.append(pltpu.async_copy(
                    ob_v.at[par],
                    out_hbm.at[_NCONT + f, wid, pl.ds(k * _CB, _CB)],
                    sem_o))
                if k + 1 < _NCHUNK:
                    nxt.wait()
            for cp in copies:
                cp.wait()
            return _

        lax.fori_loop(0, _F, cat_row, 0)

    return lookup


_REFORMAT = _make_reformat_kernel()
_LOOKUP = _make_lookup_kernel()


def kernel(x, categorical, cont_w, cont_b, bn_gamma, bn_beta, bn_mean, bn_var,
           tables):
    eps = 1e-5
    s = bn_gamma / jnp.sqrt(bn_var + eps)
    t = bn_beta - bn_mean * s
    w_fold = cont_w * s[:, None]
    c_fold = cont_w * t[:, None] + cont_b
    w_t = jnp.zeros((_D, 16), jnp.float32).at[:, :_NCONT].set(w_fold.T)
    c_t = jnp.zeros((_D, 16), jnp.float32).at[:, :_NCONT].set(c_fold.T)
    tab_t = jnp.transpose(tables, (0, 2, 1))
    cat_t = categorical.T
    x_t = x.T
    tail = jnp.zeros((_F, _D, 48), jnp.float32).at[:, :, :_VT].set(
        jnp.transpose(tables[:, _VA:, :], (0, 2, 1)))
    flat = _REFORMAT(tab_t, tail)
    out_t = _LOOKUP(cat_t, x_t, w_t, c_t, flat)
    return jnp.transpose(out_t, (2, 0, 1))
